# baseline (speedup 1.0000x reference)
import functools

import jax
import jax.numpy as jnp
from jax import lax
from jax.experimental import pallas as pl
from jax.experimental.pallas import tpu as pltpu
from jax.experimental.pallas import tpu_sc as plsc

NUM_CORES = 2
NUM_SUBCORES = 16
BUF_ROWS = 16
BLOCK_ROWS = 256


def _sc_body(table_hbm, ids_hbm, out_hbm, idx_v, rows_v, sem):
    wid = lax.axis_index("s") * NUM_CORES + lax.axis_index("c")
    rows_per_w = out_hbm.shape[0] // (NUM_CORES * NUM_SUBCORES)
    base = wid * rows_per_w
    pltpu.sync_copy(ids_hbm, idx_v)
    pltpu.async_copy(table_hbm.at[idx_v], rows_v, sem).wait()
    copies = []
    for b in range(rows_per_w // BUF_ROWS):
        dst = out_hbm.at[pl.ds(base + b * BUF_ROWS, BUF_ROWS)]
        copies.append(pltpu.async_copy(rows_v, dst, sem))
    for c in copies:
        c.wait()


def _sc_broadcast(modality_table, ids, out_rows):
    d_model = modality_table.shape[1]
    mesh = plsc.VectorSubcoreMesh(core_axis_name="c", subcore_axis_name="s")
    run = functools.partial(
        pl.kernel,
        mesh=mesh,
        out_type=jax.ShapeDtypeStruct((out_rows, d_model), jnp.float32),
        scratch_types=[
            pltpu.VMEM((BUF_ROWS,), jnp.int32),
            pltpu.VMEM((BUF_ROWS, d_model), jnp.float32),
            pltpu.SemaphoreType.DMA,
        ],
    )(_sc_body)
    return run(modality_table, ids)


def _tc_block_body(tid_ref, table_ref, out_ref):
    tid = tid_ref[0]
    r0 = table_ref[0, :]
    r1 = table_ref[1, :]
    r2 = table_ref[2, :]
    row = jnp.where(tid == 0, r0, jnp.where(tid == 1, r1, r2))
    out_ref[...] = jnp.broadcast_to(row[None, :], out_ref.shape)


def _tc_broadcast(modality_table, tid, out_rows):
    d_model = modality_table.shape[1]
    grid = (out_rows // BLOCK_ROWS,)
    return pl.pallas_call(
        _tc_block_body,
        grid_spec=pltpu.PrefetchScalarGridSpec(
            num_scalar_prefetch=1,
            grid=grid,
            in_specs=[
                pl.BlockSpec(modality_table.shape, lambda i, tid: (0, 0)),
            ],
            out_specs=pl.BlockSpec((BLOCK_ROWS, d_model), lambda i, tid: (i, 0)),
        ),
        out_shape=jax.ShapeDtypeStruct((out_rows, d_model), jnp.float32),
    )(tid, modality_table)


SC_ROWS = 2048


def kernel(embeddings, modality_table, token_type_id):
    seq_len = embeddings.shape[1]
    tid = jnp.asarray(token_type_id, dtype=jnp.int32).reshape((1,))
    ids = jnp.full((BUF_ROWS,), token_type_id, dtype=jnp.int32)
    tc_part = _tc_broadcast(modality_table, tid, seq_len - SC_ROWS)
    sc_part = _sc_broadcast(modality_table, ids, SC_ROWS)
    return jnp.concatenate([tc_part, sc_part], axis=0)


# speedup vs baseline: 4.6539x; 4.6539x over previous
import functools

import jax
import jax.numpy as jnp
from jax import lax
from jax.experimental import pallas as pl
from jax.experimental.pallas import tpu as pltpu
from jax.experimental.pallas import tpu_sc as plsc

NUM_CORES = 2
NUM_SUBCORES = 16
BUF_ROWS = 16
BLOCK_ROWS = 256


def _sc_body(table_hbm, ids_hbm, out_hbm, idx_v, rows_v, sem):
    wid = lax.axis_index("s") * NUM_CORES + lax.axis_index("c")
    rows_per_w = out_hbm.shape[0] // (NUM_CORES * NUM_SUBCORES)
    base = wid * rows_per_w
    pltpu.sync_copy(ids_hbm, idx_v)
    pltpu.async_copy(table_hbm.at[idx_v], rows_v, sem).wait()
    copies = []
    for b in range(rows_per_w // BUF_ROWS):
        dst = out_hbm.at[pl.ds(base + b * BUF_ROWS, BUF_ROWS)]
        copies.append(pltpu.async_copy(rows_v, dst, sem))
    for c in copies:
        c.wait()


def _sc_broadcast(modality_table, ids, out_rows):
    d_model = modality_table.shape[1]
    mesh = plsc.VectorSubcoreMesh(core_axis_name="c", subcore_axis_name="s")
    run = functools.partial(
        pl.kernel,
        mesh=mesh,
        out_type=jax.ShapeDtypeStruct((out_rows, d_model), jnp.float32),
        scratch_types=[
            pltpu.VMEM((BUF_ROWS,), jnp.int32),
            pltpu.VMEM((BUF_ROWS, d_model), jnp.float32),
            pltpu.SemaphoreType.DMA,
        ],
    )(_sc_body)
    return run(modality_table, ids)


def _tc_block_body(tid_ref, table_ref, out_ref):
    tid = tid_ref[0]
    r0 = table_ref[0, :]
    r1 = table_ref[1, :]
    r2 = table_ref[2, :]
    row = jnp.where(tid == 0, r0, jnp.where(tid == 1, r1, r2))
    out_ref[...] = jnp.broadcast_to(row[None, :], out_ref.shape)


def _tc_broadcast(modality_table, tid, out_rows):
    d_model = modality_table.shape[1]
    grid = (out_rows // BLOCK_ROWS,)
    return pl.pallas_call(
        _tc_block_body,
        grid_spec=pltpu.PrefetchScalarGridSpec(
            num_scalar_prefetch=1,
            grid=grid,
            in_specs=[
                pl.BlockSpec(modality_table.shape, lambda i, tid: (0, 0)),
            ],
            out_specs=pl.BlockSpec((BLOCK_ROWS, d_model), lambda i, tid: (i, 0)),
        ),
        out_shape=jax.ShapeDtypeStruct((out_rows, d_model), jnp.float32),
    )(tid, modality_table)


DMA_ROWS = 256


def _tc_dma_body(tid_ref, table_ref, out_hbm, buf, sem):
    tid = tid_ref[0]
    r0 = table_ref[0, :]
    r1 = table_ref[1, :]
    r2 = table_ref[2, :]
    row = jnp.where(tid == 0, r0, jnp.where(tid == 1, r1, r2))
    buf[...] = jnp.broadcast_to(row[None, :], buf.shape)
    copies = []
    for i in range(out_hbm.shape[0] // DMA_ROWS):
        c = pltpu.make_async_copy(buf, out_hbm.at[pl.ds(i * DMA_ROWS, DMA_ROWS)], sem)
        c.start()
        copies.append(c)
    for c in copies:
        c.wait()


def _tc_dma_broadcast(modality_table, tid, out_rows):
    d_model = modality_table.shape[1]
    return pl.pallas_call(
        _tc_dma_body,
        grid_spec=pltpu.PrefetchScalarGridSpec(
            num_scalar_prefetch=1,
            grid=(1,),
            in_specs=[pl.BlockSpec(modality_table.shape, lambda i, tid: (0, 0))],
            out_specs=pl.BlockSpec(memory_space=pl.ANY),
            scratch_shapes=[
                pltpu.VMEM((DMA_ROWS, d_model), jnp.float32),
                pltpu.SemaphoreType.DMA,
            ],
        ),
        out_shape=jax.ShapeDtypeStruct((out_rows, d_model), jnp.float32),
    )(tid, modality_table)


def kernel(embeddings, modality_table, token_type_id):
    seq_len = embeddings.shape[1]
    tid = jnp.asarray(token_type_id, dtype=jnp.int32).reshape((1,))
    return _tc_dma_broadcast(modality_table, tid, seq_len)


# speedup vs baseline: 4.7545x; 1.0216x over previous
import jax
import jax.numpy as jnp
from jax.experimental import pallas as pl
from jax.experimental.pallas import tpu as pltpu

BLOCK_ROWS = 256


def _body(tid_ref, table_ref, out_ref):
    tid = tid_ref[0]
    r0 = table_ref[0, :]
    r1 = table_ref[1, :]
    r2 = table_ref[2, :]
    row = jnp.where(tid == 0, r0, jnp.where(tid == 1, r1, r2))
    out_ref[...] = jnp.broadcast_to(row[None, :], out_ref.shape)


def kernel(embeddings, modality_table, token_type_id):
    seq_len = embeddings.shape[1]
    d_model = modality_table.shape[1]
    tid = jnp.asarray(token_type_id, dtype=jnp.int32).reshape((1,))
    return pl.pallas_call(
        _body,
        grid_spec=pltpu.PrefetchScalarGridSpec(
            num_scalar_prefetch=1,
            grid=(seq_len // BLOCK_ROWS,),
            in_specs=[
                pl.BlockSpec(modality_table.shape, lambda i, tid: (0, 0)),
            ],
            out_specs=pl.BlockSpec((BLOCK_ROWS, d_model), lambda i, tid: (i, 0)),
        ),
        out_shape=jax.ShapeDtypeStruct((seq_len, d_model), jnp.float32),
    )(tid, modality_table)
